# baseline (speedup 1.0000x reference)
import functools

import jax
import jax.numpy as jnp
from jax import lax
from jax.experimental import pallas as pl
from jax.experimental.pallas import tpu as pltpu
from jax.experimental.pallas import tpu_sc as plsc

_BINS = 30
_ROWS = 16384
_COLS = 1024

_P = 4
_GROWS = _ROWS // _P
_PROWS = _GROWS // 2

_NC = 2
_NS = 16
_L = 16
_NW = _NC * _NS
_WROWS = _PROWS // _NW
_CR = 32
_NCHUNK = _WROWS // _CR
_VPC = _CR * _COLS // _L
_UNROLL = 8

_BR = 256
_GB = _GROWS // _BR

_PB = 32
_ACC = _L * _PB


def _prep_body(p_ref, t_ref, w_ref):
    p = p_ref[...]
    t = t_ref[...]
    g = jnp.abs(p - t)
    binv = jnp.minimum(g * float(_BINS), float(_BINS - 1)).astype(jnp.uint32)
    log_p = jnp.maximum(jnp.log(p), -100.0)
    log_1mp = jnp.log(1.0 - p)
    u = log_1mp + t * (log_p - log_1mp)
    q = jnp.minimum(0.5 - 16.0 * u, 2047.0).astype(jnp.uint32)
    code = (binv << jnp.uint32(11)) | q
    lo = code[: _BR // 2, :]
    hi = code[_BR // 2 :, :]
    w_ref[...] = lax.bitcast_convert_type(lo | (hi << jnp.uint32(16)), jnp.int32)


def _make_prep(grp):
    return pl.pallas_call(
        _prep_body,
        grid=(_GB,),
        in_specs=[
            pl.BlockSpec((_BR, _COLS), lambda i, g=grp: (i + g * _GB, 0))
        ] * 2,
        out_specs=pl.BlockSpec((_BR // 2, _COLS), lambda i: (i, 0)),
        out_shape=jax.ShapeDtypeStruct((_PROWS, _COLS), jnp.int32),
    )


_preps = [_make_prep(g) for g in range(_P)]


@functools.partial(
    pl.kernel,
    mesh=plsc.VectorSubcoreMesh(core_axis_name="c", subcore_axis_name="s"),
    compiler_params=pltpu.CompilerParams(needs_layout_passes=False),
    out_type=(
        jax.ShapeDtypeStruct((_NW * _L, _PB), jnp.float32),
        jax.ShapeDtypeStruct((_NW * _L, _PB), jnp.float32),
    ),
    scratch_types=[
        pltpu.VMEM((2, _CR, _COLS), jnp.int32),
        pltpu.VMEM((_BINS * _L,), jnp.float32),
        pltpu.VMEM((_BINS * _L,), jnp.float32),
        pltpu.VMEM((_L, _PB), jnp.float32),
        pltpu.VMEM((_L, _PB), jnp.float32),
        pltpu.SemaphoreType.DMA,
        pltpu.SemaphoreType.DMA,
    ],
)
def _sc_hist(w_hbm, cnt_out, sum_out, buf, cnt_acc, sum_acc, cnt_tr, sum_tr,
             sem0, sem1):
    wid = lax.axis_index("s") * _NC + lax.axis_index("c")
    row0 = wid * _WROWS
    lane = lax.iota(jnp.int32, _L)
    ones = jnp.ones((_L,), jnp.float32)
    zeros = jnp.zeros((_L,), jnp.float32)
    sems = (sem0, sem1)
    for v in range(_BINS):
        cnt_acc[pl.ds(v * _L, _L)] = zeros
        sum_acc[pl.ds(v * _L, _L)] = zeros
    for v in range(_L):
        for h in range(_PB // _L):
            cnt_tr[v, pl.ds(h * _L, _L)] = zeros
            sum_tr[v, pl.ds(h * _L, _L)] = zeros

    def _issue(c, slot):
        pltpu.async_copy(
            w_hbm.at[pl.ds(row0 + c * _CR, _CR), :], buf.at[slot], sems[slot]
        )

    def _wait(slot):
        pltpu.make_async_copy(
            w_hbm.at[pl.ds(row0, _CR), :], buf.at[slot], sems[slot]
        ).wait()

    _issue(0, 0)
    _issue(1, 1)

    def chunk_pair(c0, carry):
        for slot in range(2):
            c = c0 + slot
            _wait(slot)

            def vec_body(o, carry2):
                gpr = _COLS // (_L * _UNROLL)
                r = lax.shift_right_logical(o, gpr.bit_length() - 1)
                cb = (o & (gpr - 1)) * (_L * _UNROLL)
                ws = [
                    buf[slot, r, pl.ds(cb + k * _L, _L)] for k in range(_UNROLL)
                ]
                for k in range(_UNROLL):
                    w = ws[k]
                    w16 = lax.shift_right_logical(w, 16)
                    wlo = w & 0xFFFF
                    bv_lo = (lax.shift_right_logical(wlo, 7) & 0x1F0) + lane
                    bv_hi = (lax.shift_right_logical(w, 23) & 0x1F0) + lane
                    q_lo = (wlo & 0x7FF).astype(jnp.float32)
                    q_hi = (w16 & 0x7FF).astype(jnp.float32)
                    plsc.addupdate_scatter(cnt_acc, [bv_lo], ones)
                    plsc.addupdate_scatter(cnt_acc, [bv_hi], ones)
                    plsc.addupdate_scatter(sum_acc, [bv_lo], q_lo)
                    plsc.addupdate_scatter(sum_acc, [bv_hi], q_hi)
                return carry2

            lax.fori_loop(0, _VPC // _UNROLL, vec_body, 0)

            @pl.when(c + 2 < _NCHUNK)
            def _():
                _issue(c + 2, slot)
        return carry

    lax.fori_loop(0, _NCHUNK // 2, lambda i, cr: chunk_pair(i * 2, cr), 0)
    for b in range(_BINS):
        bcol = jnp.full((_L,), b, jnp.int32)
        plsc.store_scatter(cnt_tr, [lane, bcol], cnt_acc[pl.ds(b * _L, _L)])
        plsc.store_scatter(sum_tr, [lane, bcol], sum_acc[pl.ds(b * _L, _L)])
    pltpu.sync_copy(cnt_tr, cnt_out.at[pl.ds(wid * _L, _L), :])
    pltpu.sync_copy(sum_tr, sum_out.at[pl.ds(wid * _L, _L), :])


def _combine_body(*refs):
    cnt_refs = refs[:_P]
    sum_refs = refs[_P : 2 * _P]
    out_ref = refs[2 * _P]
    cnt = sum(jnp.sum(r[...], axis=0, keepdims=True) for r in cnt_refs)
    s = sum(jnp.sum(r[...], axis=0, keepdims=True) for r in sum_refs)
    s = s * (1.0 / 16.0)
    nonempty = cnt > 0.0
    n = jnp.sum(nonempty.astype(jnp.float32))
    terms = jnp.where(nonempty, s / jnp.maximum(cnt, 1.0), 0.0)
    out_ref[0, 0] = jnp.where(n > 0.0, jnp.sum(terms) / jnp.maximum(n, 1.0), 0.0)


_combine = pl.pallas_call(
    _combine_body,
    in_specs=[pl.BlockSpec((_NW * _L, _PB), lambda: (0, 0))] * (2 * _P),
    out_specs=pl.BlockSpec(memory_space=pltpu.SMEM),
    out_shape=jax.ShapeDtypeStruct((1, 1), jnp.float32),
)


def kernel(pred, target, batch_size):
    cnts, sums = [], []
    for g in range(_P):
        packed = _preps[g](pred, target)
        cnt, s = _sc_hist(packed)
        cnts.append(cnt)
        sums.append(s)
    out = _combine(*cnts, *sums)
    return out[0, 0]


# speedup vs baseline: 1.1154x; 1.1154x over previous
import functools

import jax
import jax.numpy as jnp
from jax import lax
from jax.experimental import pallas as pl
from jax.experimental.pallas import tpu as pltpu
from jax.experimental.pallas import tpu_sc as plsc

_BINS = 30
_ROWS = 16384
_COLS = 1024

_P = 4
_GROWS = _ROWS // _P
_PROWS = _GROWS // 2

_NC = 2
_NS = 16
_L = 16
_NW = _NC * _NS
_WROWS = _PROWS // _NW
_CR = 32
_NCHUNK = _WROWS // _CR
_VPC = _CR * _COLS // _L
_UNROLL = 8

_BR = 512
_GB = _GROWS // _BR

_PB = 32
_ACC = _L * _PB


def _prep_body(p_ref, t_ref, w_ref):
    p = p_ref[...]
    t = t_ref[...]
    g = jnp.abs(p - t)
    binv = jnp.minimum(g * float(_BINS), float(_BINS - 1)).astype(jnp.uint32)
    log_p = jnp.maximum(jnp.log(p), -100.0)
    log_1mp = jnp.log(1.0 - p)
    u = log_1mp + t * (log_p - log_1mp)
    q = jnp.minimum(0.5 - 16.0 * u, 2047.0).astype(jnp.uint32)
    code = (binv << jnp.uint32(11)) | q
    lo = code[: _BR // 2, :]
    hi = code[_BR // 2 :, :]
    w_ref[...] = lax.bitcast_convert_type(lo | (hi << jnp.uint32(16)), jnp.int32)


def _make_prep(grp):
    return pl.pallas_call(
        _prep_body,
        grid=(_GB,),
        in_specs=[
            pl.BlockSpec((_BR, _COLS), lambda i, g=grp: (i + g * _GB, 0))
        ] * 2,
        out_specs=pl.BlockSpec((_BR // 2, _COLS), lambda i: (i, 0)),
        out_shape=jax.ShapeDtypeStruct((_PROWS, _COLS), jnp.int32),
    )


_preps = [_make_prep(g) for g in range(_P)]


@functools.partial(
    pl.kernel,
    mesh=plsc.VectorSubcoreMesh(core_axis_name="c", subcore_axis_name="s"),
    compiler_params=pltpu.CompilerParams(needs_layout_passes=False),
    out_type=(
        jax.ShapeDtypeStruct((_NW * _L, _PB), jnp.float32),
        jax.ShapeDtypeStruct((_NW * _L, _PB), jnp.float32),
    ),
    scratch_types=[
        pltpu.VMEM((2, _CR, _COLS), jnp.int32),
        pltpu.VMEM((_BINS * _L,), jnp.float32),
        pltpu.VMEM((_BINS * _L,), jnp.float32),
        pltpu.VMEM((_L, _PB), jnp.float32),
        pltpu.VMEM((_L, _PB), jnp.float32),
        pltpu.SemaphoreType.DMA,
        pltpu.SemaphoreType.DMA,
    ],
)
def _sc_hist(w_hbm, cnt_out, sum_out, buf, cnt_acc, sum_acc, cnt_tr, sum_tr,
             sem0, sem1):
    wid = lax.axis_index("s") * _NC + lax.axis_index("c")
    row0 = wid * _WROWS
    lane = lax.iota(jnp.int32, _L)
    ones = jnp.ones((_L,), jnp.float32)
    zeros = jnp.zeros((_L,), jnp.float32)
    sems = (sem0, sem1)
    for v in range(_BINS):
        cnt_acc[pl.ds(v * _L, _L)] = zeros
        sum_acc[pl.ds(v * _L, _L)] = zeros
    for v in range(_L):
        for h in range(_PB // _L):
            cnt_tr[v, pl.ds(h * _L, _L)] = zeros
            sum_tr[v, pl.ds(h * _L, _L)] = zeros

    def _issue(c, slot):
        pltpu.async_copy(
            w_hbm.at[pl.ds(row0 + c * _CR, _CR), :], buf.at[slot], sems[slot]
        )

    def _wait(slot):
        pltpu.make_async_copy(
            w_hbm.at[pl.ds(row0, _CR), :], buf.at[slot], sems[slot]
        ).wait()

    _issue(0, 0)
    _issue(1, 1)

    def chunk_pair(c0, carry):
        for slot in range(2):
            c = c0 + slot
            _wait(slot)

            def vec_body(o, carry2):
                gpr = _COLS // (_L * _UNROLL)
                r = lax.shift_right_logical(o, gpr.bit_length() - 1)
                cb = (o & (gpr - 1)) * (_L * _UNROLL)
                ws = [
                    buf[slot, r, pl.ds(cb + k * _L, _L)] for k in range(_UNROLL)
                ]
                for k in range(_UNROLL):
                    w = ws[k]
                    bv_lo = (lax.shift_right_logical(w, 7) & 0x1F0) + lane
                    bv_hi = (lax.shift_right_logical(w, 23) & 0x1F0) + lane
                    q_lo = (w & 0x7FF).astype(jnp.float32)
                    q_hi = (lax.shift_right_logical(w, 16) & 0x7FF).astype(
                        jnp.float32
                    )
                    plsc.addupdate_scatter(cnt_acc, [bv_lo], ones)
                    plsc.addupdate_scatter(cnt_acc, [bv_hi], ones)
                    plsc.addupdate_scatter(sum_acc, [bv_lo], q_lo)
                    plsc.addupdate_scatter(sum_acc, [bv_hi], q_hi)
                return carry2

            lax.fori_loop(0, _VPC // _UNROLL, vec_body, 0)

            @pl.when(c + 2 < _NCHUNK)
            def _():
                _issue(c + 2, slot)
        return carry

    lax.fori_loop(0, _NCHUNK // 2, lambda i, cr: chunk_pair(i * 2, cr), 0)
    for b in range(_BINS):
        bcol = jnp.full((_L,), b, jnp.int32)
        plsc.store_scatter(cnt_tr, [lane, bcol], cnt_acc[pl.ds(b * _L, _L)])
        plsc.store_scatter(sum_tr, [lane, bcol], sum_acc[pl.ds(b * _L, _L)])
    pltpu.sync_copy(cnt_tr, cnt_out.at[pl.ds(wid * _L, _L), :])
    pltpu.sync_copy(sum_tr, sum_out.at[pl.ds(wid * _L, _L), :])


def _combine_body(*refs):
    cnt_refs = refs[:_P]
    sum_refs = refs[_P : 2 * _P]
    out_ref = refs[2 * _P]
    cnt = sum(jnp.sum(r[...], axis=0, keepdims=True) for r in cnt_refs)
    s = sum(jnp.sum(r[...], axis=0, keepdims=True) for r in sum_refs)
    s = s * (1.0 / 16.0)
    nonempty = cnt > 0.0
    n = jnp.sum(nonempty.astype(jnp.float32))
    terms = jnp.where(nonempty, s / jnp.maximum(cnt, 1.0), 0.0)
    out_ref[0, 0] = jnp.where(n > 0.0, jnp.sum(terms) / jnp.maximum(n, 1.0), 0.0)


_combine = pl.pallas_call(
    _combine_body,
    in_specs=[pl.BlockSpec((_NW * _L, _PB), lambda: (0, 0))] * (2 * _P),
    out_specs=pl.BlockSpec(memory_space=pltpu.SMEM),
    out_shape=jax.ShapeDtypeStruct((1, 1), jnp.float32),
)


def kernel(pred, target, batch_size):
    cnts, sums = [], []
    for g in range(_P):
        packed = _preps[g](pred, target)
        cnt, s = _sc_hist(packed)
        cnts.append(cnt)
        sums.append(s)
    out = _combine(*cnts, *sums)
    return out[0, 0]
